# Initial kernel scaffold; baseline (speedup 1.0000x reference)
import jax
import jax.numpy as jnp
from jax.experimental import pallas as pl


def kernel(h, g, edge_idx, edge_type):
    raise NotImplementedError("write your pallas kernel here")


# speedup vs baseline: 2.0708x; 2.0708x over previous
import functools
import jax
import jax.numpy as jnp
from jax import lax
from jax.experimental import pallas as pl
from jax.experimental.pallas import tpu as pltpu, tpu_sc as plsc

D = 128
C = 128
L = 16
NC = 2
NS = 16
NW = NC * NS

_PERM_DNUMS = lax.GatherDimensionNumbers(
    offset_dims=(), collapsed_slice_dims=(0,), start_index_map=(0,))


def _lane_perm(v, idx):
  return lax.gather(v, idx[:, None], _PERM_DNUMS, (1,),
                    mode=lax.GatherScatterMode.PROMISE_IN_BOUNDS)


def _body(h_hbm, g_hbm, row_hbm, col_hbm, typ_hbm, out_hbm,
          row_v, col_v, typ_v, hr_v, hc_v, gr_v, o_v, sem, *, n_chunks):
  wid = lax.axis_index("s") * NC + lax.axis_index("c")
  lane = lax.iota(jnp.int32, L)

  def chunk_body(i, carry):
    base = pl.multiple_of((i * NW + wid) * C, C)
    pltpu.sync_copy(row_hbm.at[pl.ds(base, C)], row_v)
    pltpu.sync_copy(col_hbm.at[pl.ds(base, C)], col_v)
    pltpu.sync_copy(typ_hbm.at[pl.ds(base, C)], typ_v)
    d1 = pltpu.async_copy(h_hbm.at[row_v], hr_v, sem)
    d2 = pltpu.async_copy(h_hbm.at[col_v], hc_v, sem)
    d3 = pltpu.async_copy(g_hbm.at[typ_v], gr_v, sem)
    d1.wait()
    d2.wait()
    d3.wait()

    def group(gi, carry2):
      res = jnp.zeros((L,), jnp.float32)
      for t in range(L):
        e = gi * L + t
        acc = jnp.zeros((L,), jnp.float32)
        for j in range(D // L):
          sl = pl.ds(j * L, L)
          acc = acc + jnp.abs(hr_v[e, sl] + gr_v[e, sl] - hc_v[e, sl])
        for dist in (8, 4, 2, 1):
          acc = acc + _lane_perm(acc, lane ^ dist)
        res = jnp.where(lane == t, acc, res)
      o_v[pl.ds(pl.multiple_of(gi * L, L), L)] = res
      return carry2

    lax.fori_loop(0, C // L, group, 0, unroll=False)
    pltpu.sync_copy(o_v, out_hbm.at[pl.ds(base, C)])
    return carry

  lax.fori_loop(0, n_chunks, chunk_body, 0, unroll=False)


def kernel(h, g, edge_idx, edge_type):
  E = edge_idx.shape[1]
  per_round = NW * C
  n_chunks = -(-E // per_round)
  e_pad = n_chunks * per_round
  pad = e_pad - E
  row = jnp.pad(edge_idx[0], (0, pad))
  col = jnp.pad(edge_idx[1], (0, pad))
  typ = jnp.pad(edge_type, (0, pad))

  mesh = plsc.VectorSubcoreMesh(core_axis_name="c", subcore_axis_name="s")
  kfn = pl.kernel(
      functools.partial(_body, n_chunks=n_chunks),
      out_type=jax.ShapeDtypeStruct((e_pad,), jnp.float32),
      mesh=mesh,
      scratch_types=[
          pltpu.VMEM((C,), jnp.int32),
          pltpu.VMEM((C,), jnp.int32),
          pltpu.VMEM((C,), jnp.int32),
          pltpu.VMEM((C, D), jnp.float32),
          pltpu.VMEM((C, D), jnp.float32),
          pltpu.VMEM((C, D), jnp.float32),
          pltpu.VMEM((C,), jnp.float32),
          pltpu.SemaphoreType.DMA,
      ],
  )
  out = kfn(h, g, row, col, typ)
  return out[:E]
